# baseline (speedup 1.0000x reference)
import functools

import jax
import jax.numpy as jnp
from jax import lax
from jax.experimental import pallas as pl
from jax.experimental.pallas import tpu as pltpu
from jax.experimental.pallas import tpu_sc as plsc

DIM = 64
LANES = 16
BLK = 128
CHUNK = 512
N_FULL_CHUNKS = 1953
MAIN_COLS = N_FULL_CHUNKS * CHUNK
CHUNKS_PER_W = 61
B = 16384
PAD_ROW = B
GATH_ROWS = B + 8
TAIL_OFF = 8 * CHUNK

_params = pltpu.CompilerParams(needs_layout_passes=False, use_tc_tiling_on_sc=True)


def _make_mesh():
    return plsc.VectorSubcoreMesh(core_axis_name="c", subcore_axis_name="s")


def kernel(centers, contexts, in_embed, out_embed):
    V = in_embed.shape[0]
    c = centers.astype(jnp.int32)
    x = contexts.astype(jnp.int32)

    ord_c = jnp.argsort(c).astype(jnp.int32)
    sc = c[ord_c]
    ord_x = jnp.argsort(x).astype(jnp.int32)
    sx = x[ord_x]
    edges = jnp.minimum(jnp.arange(2048, dtype=jnp.int32) * CHUNK, MAIN_COLS)
    seg_c = jnp.searchsorted(sc, edges).astype(jnp.int32)
    seg_x = jnp.searchsorted(sx, edges).astype(jnp.int32)

    tV = in_embed.T
    tU = out_embed.T
    tailV = jnp.pad(tV[:, MAIN_COLS:], ((0, 0), (0, BLK - (V - MAIN_COLS))))
    tailU = jnp.pad(tU[:, MAIN_COLS:], ((0, 0), (0, BLK - (V - MAIN_COLS))))

    gath = jax.ShapeDtypeStruct((GATH_ROWS, BLK), jnp.float32)

    @functools.partial(
        pl.kernel,
        out_type=(gath, gath),
        mesh=_make_mesh(),
        compiler_params=_params,
        scratch_types=[
            pltpu.VMEM((8, TAIL_OFF + 8 * BLK), jnp.float32),
            pltpu.VMEM((B,), jnp.int32),
            pltpu.VMEM((B,), jnp.int32),
            pltpu.VMEM((2048,), jnp.int32),
            pltpu.VMEM((LANES, BLK), jnp.float32),
            pltpu.VMEM((LANES,), jnp.int32),
            pltpu.SemaphoreType.DMA,
            pltpu.SemaphoreType.DMA,
        ],
    )
    def _scan_extract(tv_h, tu_h, tlv_h, tlu_h, sc_h, oc_h, gc_h, sx_h, ox_h, gx_h,
                      vg, ug, slab, sidx, sord, sseg, rowbuf, dsti, sem, sem2):
        wid = lax.axis_index("s") * 2 + lax.axis_index("c")
        iota = lax.iota(jnp.int32, LANES)
        sv8 = iota % 8
        k2v = iota // 8

        def gscal(ref, pos):
            return plsc.load_gather(ref, [jnp.zeros((LANES,), jnp.int32) + pos])[0]

        for tab, tail_tab, sh, oh, gh, gout in (
            (tv_h, tlv_h, sc_h, oc_h, gc_h, vg),
            (tu_h, tlu_h, sx_h, ox_h, gx_h, ug),
        ):
            pltpu.sync_copy(sh, sidx)
            pltpu.sync_copy(oh, sord)
            pltpu.sync_copy(gh, sseg)

            def extract(s0, s1, col0, base, mult):
                n_grp = (s1 - s0 + LANES - 1) // LANES

                def grp(m, carry):
                    p0 = s0 + m * LANES
                    pv = jnp.minimum(p0 + iota, B - 1)
                    valid = (p0 + iota) < s1
                    rvec = plsc.load_gather(sidx, [pv])
                    dvec = plsc.load_gather(sord, [pv])
                    cvec = jnp.clip(rvec - col0, 0, mult - 1)
                    dvec = jnp.where(valid, dvec, PAD_ROW)
                    for l in range(LANES):
                        cl = cvec[l]
                        for j in range(DIM // LANES):
                            colv = base + (2 * j + k2v) * mult + cl
                            rowbuf[l, pl.ds(16 * j, 16)] = plsc.load_gather(slab, [sv8, colv])
                    dsti[pl.ds(0, LANES)] = dvec
                    pltpu.async_copy(rowbuf, gout.at[dsti], sem2).wait()
                    return carry

                lax.fori_loop(0, n_grp, grp, 0)

            def stream_chunk(col0):
                for k in range(8):
                    pltpu.async_copy(
                        tab.at[pl.ds(8 * k, 8), pl.ds(col0, CHUNK)],
                        slab.at[pl.ds(0, 8), pl.ds(k * CHUNK, CHUNK)], sem)
                for k in range(8):
                    pltpu.make_async_copy(
                        tab.at[pl.ds(0, 8), pl.ds(0, CHUNK)],
                        slab.at[pl.ds(0, 8), pl.ds(k * CHUNK, CHUNK)], sem).wait()

            def chunk_body(g):
                col0 = pl.multiple_of(g * CHUNK, CHUNK)
                stream_chunk(col0)
                s0 = gscal(sseg, g)
                s1 = gscal(sseg, g + 1)
                extract(s0, s1, col0, 0, CHUNK)

            def main_loop(ci, carry):
                chunk_body(wid * CHUNKS_PER_W + ci)
                return carry

            lax.fori_loop(0, CHUNKS_PER_W, main_loop, 0)

            @pl.when(wid == 31)
            def _tail():
                chunk_body(N_FULL_CHUNKS - 1)
                for k in range(8):
                    pltpu.async_copy(
                        tail_tab.at[pl.ds(8 * k, 8), pl.ds(0, BLK)],
                        slab.at[pl.ds(0, 8), pl.ds(TAIL_OFF + k * BLK, BLK)], sem)
                for k in range(8):
                    pltpu.make_async_copy(
                        tail_tab.at[pl.ds(0, 8), pl.ds(0, BLK)],
                        slab.at[pl.ds(0, 8), pl.ds(TAIL_OFF + k * BLK, BLK)], sem).wait()
                s0 = gscal(sseg, N_FULL_CHUNKS)
                extract(s0, B, MAIN_COLS, TAIL_OFF, BLK)

    vg, ug = _scan_extract(tV, tU, tailV, tailU, sc, ord_c, seg_c, sx, ord_x, seg_x)

    @functools.partial(
        pl.kernel,
        out_type=jax.ShapeDtypeStruct((B,), jnp.float32),
        mesh=_make_mesh(),
        compiler_params=_params,
        scratch_types=[
            pltpu.VMEM((128, BLK), jnp.float32),
            pltpu.VMEM((128, BLK), jnp.float32),
            pltpu.VMEM((512,), jnp.float32),
            pltpu.SemaphoreType.DMA,
        ],
    )
    def _dot(vg_h, ug_h, o_h, vs, us, obuf, sem):
        wid = lax.axis_index("s") * 2 + lax.axis_index("c")
        iota = lax.iota(jnp.int32, LANES)
        base = wid * 512

        def quarter(q, carry):
            r0 = pl.multiple_of(base + q * 128, 128)
            cp1 = pltpu.async_copy(vg_h.at[pl.ds(r0, 128), pl.ds(0, BLK)], vs, sem)
            cp2 = pltpu.async_copy(ug_h.at[pl.ds(r0, 128), pl.ds(0, BLK)], us, sem)
            cp1.wait()
            cp2.wait()

            def blk(m, carry2):
                b0 = pl.multiple_of(m * LANES, LANES)
                tot = jnp.zeros((LANES,), jnp.float32)
                for r in range(LANES):
                    row = b0 + r
                    acc = vs[row, pl.ds(0, 16)] * us[row, pl.ds(0, 16)]
                    for j in range(1, DIM // LANES):
                        acc = acc + vs[row, pl.ds(16 * j, 16)] * us[row, pl.ds(16 * j, 16)]
                    tot = jnp.where(iota == r, jnp.sum(acc), tot)
                oq = pl.multiple_of(q * 128 + b0, LANES)
                obuf[pl.ds(oq, LANES)] = tot
                return carry2

            lax.fori_loop(0, 8, blk, 0)
            return carry

        lax.fori_loop(0, 4, quarter, 0)
        pltpu.sync_copy(obuf, o_h.at[pl.ds(base, 512)])

    return _dot(vg, ug)


# speedup vs baseline: 1.0158x; 1.0158x over previous
import functools

import jax
import jax.numpy as jnp
from jax import lax
from jax.experimental import pallas as pl
from jax.experimental.pallas import tpu as pltpu
from jax.experimental.pallas import tpu_sc as plsc

DIM = 64
LANES = 16
BLK = 128
CHUNK = 512
N_FULL_CHUNKS = 1953
MAIN_COLS = N_FULL_CHUNKS * CHUNK
CHUNKS_PER_W = 61
B = 16384
PAD_ROW = B
GATH_ROWS = B + 8
TAIL_OFF = 8 * CHUNK

_params = pltpu.CompilerParams(needs_layout_passes=False, use_tc_tiling_on_sc=True)
_DIAG_NO_SCATTER = True


def _make_mesh():
    return plsc.VectorSubcoreMesh(core_axis_name="c", subcore_axis_name="s")


def kernel(centers, contexts, in_embed, out_embed):
    V = in_embed.shape[0]
    c = centers.astype(jnp.int32)
    x = contexts.astype(jnp.int32)

    ord_c = jnp.argsort(c).astype(jnp.int32)
    sc = c[ord_c]
    ord_x = jnp.argsort(x).astype(jnp.int32)
    sx = x[ord_x]
    edges = jnp.minimum(jnp.arange(2048, dtype=jnp.int32) * CHUNK, MAIN_COLS)
    seg_c = jnp.searchsorted(sc, edges).astype(jnp.int32)
    seg_x = jnp.searchsorted(sx, edges).astype(jnp.int32)

    tV = in_embed.T
    tU = out_embed.T
    tailV = jnp.pad(tV[:, MAIN_COLS:], ((0, 0), (0, BLK - (V - MAIN_COLS))))
    tailU = jnp.pad(tU[:, MAIN_COLS:], ((0, 0), (0, BLK - (V - MAIN_COLS))))

    gath = jax.ShapeDtypeStruct((GATH_ROWS, BLK), jnp.float32)

    @functools.partial(
        pl.kernel,
        out_type=(gath, gath),
        mesh=_make_mesh(),
        compiler_params=_params,
        scratch_types=[
            pltpu.VMEM((8, 8 * CHUNK), jnp.float32),
            pltpu.VMEM((8, 8 * CHUNK), jnp.float32),
            pltpu.VMEM((8, 8 * BLK), jnp.float32),
            pltpu.VMEM((B,), jnp.int32),
            pltpu.VMEM((B,), jnp.int32),
            pltpu.VMEM((2048,), jnp.int32),
            pltpu.VMEM((LANES, BLK), jnp.float32),
            pltpu.VMEM((LANES, BLK), jnp.float32),
            pltpu.VMEM((LANES,), jnp.int32),
            pltpu.VMEM((LANES,), jnp.int32),
            pltpu.SemaphoreType.DMA,
            pltpu.SemaphoreType.DMA,
            pltpu.SemaphoreType.DMA,
            pltpu.SemaphoreType.DMA,
        ],
    )
    def _scan_extract(tv_h, tu_h, tlv_h, tlu_h, sc_h, oc_h, gc_h, sx_h, ox_h, gx_h,
                      vg, ug, slabA, slabB, tslab, sidx, sord, sseg,
                      rowbufA, rowbufB, dstiA, dstiB, semA, semB, semSA, semSB):
        wid = lax.axis_index("s") * 2 + lax.axis_index("c")
        iota = lax.iota(jnp.int32, LANES)
        sv8 = iota % 8
        k2v = iota // 8

        def gscal(ref, pos):
            return plsc.load_gather(ref, [jnp.zeros((LANES,), jnp.int32) + pos])[0]

        for tab, tail_tab, sh, oh, gh, gout in (
            (tv_h, tlv_h, sc_h, oc_h, gc_h, vg),
            (tu_h, tlu_h, sx_h, ox_h, gx_h, ug),
        ):
            pltpu.sync_copy(sh, sidx)
            pltpu.sync_copy(oh, sord)
            pltpu.sync_copy(gh, sseg)

            def fire_stream(g, slab, sem):
                col0 = pl.multiple_of(g * CHUNK, CHUNK)
                for k in range(8):
                    pltpu.async_copy(
                        tab.at[pl.ds(8 * k, 8), pl.ds(col0, CHUNK)],
                        slab.at[pl.ds(0, 8), pl.ds(k * CHUNK, CHUNK)], sem)

            def wait_stream(slab, sem):
                for k in range(8):
                    pltpu.make_async_copy(
                        tab.at[pl.ds(0, 8), pl.ds(0, CHUNK)],
                        slab.at[pl.ds(0, 8), pl.ds(k * CHUNK, CHUNK)], sem).wait()

            def drain_scatters(n, rowbuf, semS):
                def d(_, c):
                    pltpu.make_async_copy(
                        gout.at[pl.ds(0, LANES), pl.ds(0, BLK)], rowbuf, semS).wait()
                    return c
                lax.fori_loop(0, n, d, 0)

            def extract(slab, rowbuf, dsti, semS, pend, s0, s1, col0, base, mult):
                n_grp = (s1 - s0 + LANES - 1) // LANES

                def grp(m, p):
                    drain_scatters(p, rowbuf, semS)
                    p0 = s0 + m * LANES
                    pv = jnp.minimum(p0 + iota, B - 1)
                    valid = (p0 + iota) < s1
                    rvec = plsc.load_gather(sidx, [pv])
                    dvec = plsc.load_gather(sord, [pv])
                    cvec = jnp.clip(rvec - col0, 0, mult - 1)
                    dvec = jnp.where(valid, dvec, PAD_ROW)
                    for l in range(LANES):
                        cl = cvec[l]
                        for j in range(DIM // LANES):
                            colv = base + (2 * j + k2v) * mult + cl
                            rowbuf[l, pl.ds(16 * j, 16)] = plsc.load_gather(slab, [sv8, colv])
                    dsti[pl.ds(0, LANES)] = dvec
                    pltpu.async_copy(rowbuf, gout.at[dsti], semS)
                    return 1

                return lax.fori_loop(0, n_grp, grp, pend)

            def extract_chunk(g, slab, rowbuf, dsti, semS, pend):
                col0 = pl.multiple_of(g * CHUNK, CHUNK)
                s0 = gscal(sseg, g)
                s1 = gscal(sseg, g + 1)
                return extract(slab, rowbuf, dsti, semS, pend, s0, s1, col0, 0, CHUNK)

            g_base = wid * CHUNKS_PER_W
            fire_stream(g_base, slabA, semA)

            def pair_body(p, carry):
                pa, pb = carry
                g0 = g_base + 2 * p
                fire_stream(g0 + 1, slabB, semB)
                wait_stream(slabA, semA)
                pa = extract_chunk(g0, slabA, rowbufA, dstiA, semSA, pa)
                fire_stream(g0 + 2, slabA, semA)
                wait_stream(slabB, semB)
                pb = extract_chunk(g0 + 1, slabB, rowbufB, dstiB, semSB, pb)
                return pa, pb

            pa, pb = lax.fori_loop(0, CHUNKS_PER_W // 2, pair_body, (0, 0))
            wait_stream(slabA, semA)
            pa = extract_chunk(g_base + CHUNKS_PER_W - 1, slabA, rowbufA, dstiA, semSA, pa)
            drain_scatters(pa, rowbufA, semSA)
            drain_scatters(pb, rowbufB, semSB)

            @pl.when(wid == 31)
            def _tail():
                fire_stream(N_FULL_CHUNKS - 1, slabB, semB)
                wait_stream(slabB, semB)
                pt = extract_chunk(N_FULL_CHUNKS - 1, slabB, rowbufB, dstiB, semSB, 0)
                for k in range(8):
                    pltpu.async_copy(
                        tail_tab.at[pl.ds(8 * k, 8), pl.ds(0, BLK)],
                        tslab.at[pl.ds(0, 8), pl.ds(k * BLK, BLK)], semB)
                for k in range(8):
                    pltpu.make_async_copy(
                        tail_tab.at[pl.ds(0, 8), pl.ds(0, BLK)],
                        tslab.at[pl.ds(0, 8), pl.ds(k * BLK, BLK)], semB).wait()
                s0 = gscal(sseg, N_FULL_CHUNKS)
                pt = extract(tslab, rowbufB, dstiB, semSB, pt, s0, B, MAIN_COLS, 0, BLK)
                drain_scatters(pt, rowbufB, semSB)

    vg, ug = _scan_extract(tV, tU, tailV, tailU, sc, ord_c, seg_c, sx, ord_x, seg_x)

    @functools.partial(
        pl.kernel,
        out_type=jax.ShapeDtypeStruct((B,), jnp.float32),
        mesh=_make_mesh(),
        compiler_params=_params,
        scratch_types=[
            pltpu.VMEM((128, BLK), jnp.float32),
            pltpu.VMEM((128, BLK), jnp.float32),
            pltpu.VMEM((512,), jnp.float32),
            pltpu.SemaphoreType.DMA,
        ],
    )
    def _dot(vg_h, ug_h, o_h, vs, us, obuf, sem):
        wid = lax.axis_index("s") * 2 + lax.axis_index("c")
        iota = lax.iota(jnp.int32, LANES)
        base = wid * 512

        def quarter(q, carry):
            r0 = pl.multiple_of(base + q * 128, 128)
            cp1 = pltpu.async_copy(vg_h.at[pl.ds(r0, 128), pl.ds(0, BLK)], vs, sem)
            cp2 = pltpu.async_copy(ug_h.at[pl.ds(r0, 128), pl.ds(0, BLK)], us, sem)
            cp1.wait()
            cp2.wait()

            def blk(m, carry2):
                b0 = pl.multiple_of(m * LANES, LANES)
                tot = jnp.zeros((LANES,), jnp.float32)
                for r in range(LANES):
                    row = b0 + r
                    acc = vs[row, pl.ds(0, 16)] * us[row, pl.ds(0, 16)]
                    for j in range(1, DIM // LANES):
                        acc = acc + vs[row, pl.ds(16 * j, 16)] * us[row, pl.ds(16 * j, 16)]
                    tot = jnp.where(iota == r, jnp.sum(acc), tot)
                oq = pl.multiple_of(q * 128 + b0, LANES)
                obuf[pl.ds(oq, LANES)] = tot
                return carry2

            lax.fori_loop(0, 8, blk, 0)
            return carry

        lax.fori_loop(0, 4, quarter, 0)
        pltpu.sync_copy(obuf, o_h.at[pl.ds(base, 512)])

    return _dot(vg, ug)


# speedup vs baseline: 1.8555x; 1.8266x over previous
import functools

import jax
import jax.numpy as jnp
from jax import lax
from jax.experimental import pallas as pl
from jax.experimental.pallas import tpu as pltpu
from jax.experimental.pallas import tpu_sc as plsc

DIM = 64
LANES = 16
BLK = 128
CHUNK = 512
N_MAIN_CHUNKS = 1953
MAIN_COLS = N_MAIN_CHUNKS * CHUNK
NCHUNKS = N_MAIN_CHUNKS + 1
CHUNKS_PER_W = 61
B = 16384
GATH_ROWS = B + NCHUNKS * (LANES - 1) + 2 * LANES

_params = pltpu.CompilerParams(needs_layout_passes=False, use_tc_tiling_on_sc=True)


def _make_mesh():
    return plsc.VectorSubcoreMesh(core_axis_name="c", subcore_axis_name="s")


def kernel(centers, contexts, in_embed, out_embed):
    V = in_embed.shape[0]

    def prep(idx):
        order = jnp.argsort(idx).astype(jnp.int32)
        s = idx[order]
        edges = jnp.arange(NCHUNKS, dtype=jnp.int32) * CHUNK
        seg = jnp.searchsorted(s, edges).astype(jnp.int32)
        segf = jnp.concatenate([seg, jnp.full((1,), B, jnp.int32)])
        cnt = segf[1:] - segf[:-1]
        slots = ((cnt + LANES - 1) // LANES) * LANES
        astart = (jnp.cumsum(slots) - slots).astype(jnp.int32)
        gidx = s // CHUNK
        rowid_sorted = astart[gidx] + jnp.arange(B, dtype=jnp.int32) - segf[gidx]
        rowid = jnp.zeros((B,), jnp.int32).at[order].set(rowid_sorted)
        segp = jnp.zeros((2048,), jnp.int32).at[: NCHUNKS + 1].set(segf)
        astp = jnp.zeros((2048,), jnp.int32).at[:NCHUNKS].set(astart)
        return s, segp, astp, rowid

    sc, seg_c, ast_c, rowid_c = prep(centers.astype(jnp.int32))
    sx, seg_x, ast_x, rowid_x = prep(contexts.astype(jnp.int32))

    tV = in_embed.T
    tU = out_embed.T
    tailV = jnp.pad(tV[:, MAIN_COLS:], ((0, 0), (0, BLK - (V - MAIN_COLS))))
    tailU = jnp.pad(tU[:, MAIN_COLS:], ((0, 0), (0, BLK - (V - MAIN_COLS))))

    gath = jax.ShapeDtypeStruct((GATH_ROWS, BLK), jnp.float32)

    @functools.partial(
        pl.kernel,
        out_type=(gath, gath),
        mesh=_make_mesh(),
        compiler_params=_params,
        scratch_types=[
            pltpu.VMEM((8, 8 * CHUNK), jnp.float32),
            pltpu.VMEM((8, 8 * CHUNK), jnp.float32),
            pltpu.VMEM((8, 8 * BLK), jnp.float32),
            pltpu.VMEM((B,), jnp.int32),
            pltpu.VMEM((2048,), jnp.int32),
            pltpu.VMEM((2048,), jnp.int32),
            pltpu.VMEM((LANES, BLK), jnp.float32),
            pltpu.VMEM((LANES, BLK), jnp.float32),
            pltpu.SemaphoreType.DMA,
            pltpu.SemaphoreType.DMA,
            pltpu.SemaphoreType.DMA,
            pltpu.SemaphoreType.DMA,
        ],
    )
    def _scan_extract(tv_h, tu_h, tlv_h, tlu_h, sc_h, gc_h, ac_h, sx_h, gx_h, ax_h,
                      vg, ug, slabA, slabB, tslab, sidx, sseg, sast,
                      rowbufA, rowbufB, semA, semB, semSA, semSB):
        wid = lax.axis_index("s") * 2 + lax.axis_index("c")
        iota = lax.iota(jnp.int32, LANES)
        sv8 = iota % 8
        k2v = iota // 8

        def gscal(ref, pos):
            return plsc.load_gather(ref, [jnp.zeros((LANES,), jnp.int32) + pos])[0]

        for tab, tail_tab, sh, gh, ah, gout in (
            (tv_h, tlv_h, sc_h, gc_h, ac_h, vg),
            (tu_h, tlu_h, sx_h, gx_h, ax_h, ug),
        ):
            pltpu.sync_copy(sh, sidx)
            pltpu.sync_copy(gh, sseg)
            pltpu.sync_copy(ah, sast)

            def fire_stream(g, slab, sem):
                col0 = pl.multiple_of(g * CHUNK, CHUNK)
                for k in range(8):
                    pltpu.async_copy(
                        tab.at[pl.ds(8 * k, 8), pl.ds(col0, CHUNK)],
                        slab.at[pl.ds(0, 8), pl.ds(k * CHUNK, CHUNK)], sem)

            def wait_stream(slab, sem):
                for k in range(8):
                    pltpu.make_async_copy(
                        tab.at[pl.ds(0, 8), pl.ds(0, CHUNK)],
                        slab.at[pl.ds(0, 8), pl.ds(k * CHUNK, CHUNK)], sem).wait()

            def drain_writes(n, rowbuf, semS):
                def d(_, cc):
                    pltpu.make_async_copy(
                        gout.at[pl.ds(0, LANES), pl.ds(0, BLK)], rowbuf, semS).wait()
                    return cc
                lax.fori_loop(0, n, d, 0)

            def extract(slab, rowbuf, semS, pend, s0, s1, abase, col0, base, mult):
                n_grp = (s1 - s0 + LANES - 1) // LANES

                def grp(m, p):
                    drain_writes(p, rowbuf, semS)
                    p0 = s0 + m * LANES
                    pv = jnp.minimum(p0 + iota, B - 1)
                    rvec = plsc.load_gather(sidx, [pv])
                    cvec = jnp.clip(rvec - col0, 0, mult - 1)
                    for l in range(LANES):
                        cl = cvec[l]
                        for j in range(DIM // LANES):
                            colv = base + (2 * j + k2v) * mult + cl
                            rowbuf[l, pl.ds(16 * j, 16)] = plsc.load_gather(slab, [sv8, colv])
                    arow = pl.multiple_of(abase + m * LANES, LANES)
                    pltpu.async_copy(
                        rowbuf, gout.at[pl.ds(arow, LANES), pl.ds(0, BLK)], semS)
                    return 1

                return lax.fori_loop(0, n_grp, grp, pend)

            def extract_chunk(g, slab, rowbuf, semS, pend):
                col0 = pl.multiple_of(g * CHUNK, CHUNK)
                s0 = gscal(sseg, g)
                s1 = gscal(sseg, g + 1)
                abase = pl.multiple_of(gscal(sast, g), LANES)
                return extract(slab, rowbuf, semS, pend, s0, s1, abase, col0, 0, CHUNK)

            g_base = wid * CHUNKS_PER_W
            fire_stream(g_base, slabA, semA)

            def pair_body(p, carry):
                pa, pb = carry
                g0 = g_base + 2 * p
                fire_stream(g0 + 1, slabB, semB)
                wait_stream(slabA, semA)
                pa = extract_chunk(g0, slabA, rowbufA, semSA, pa)
                fire_stream(g0 + 2, slabA, semA)
                wait_stream(slabB, semB)
                pb = extract_chunk(g0 + 1, slabB, rowbufB, semSB, pb)
                return pa, pb

            pa, pb = lax.fori_loop(0, CHUNKS_PER_W // 2, pair_body, (0, 0))
            wait_stream(slabA, semA)
            pa = extract_chunk(g_base + CHUNKS_PER_W - 1, slabA, rowbufA, semSA, pa)
            drain_writes(pa, rowbufA, semSA)
            drain_writes(pb, rowbufB, semSB)

            @pl.when(wid == 31)
            def _tail():
                fire_stream(N_MAIN_CHUNKS - 1, slabB, semB)
                wait_stream(slabB, semB)
                pt = extract_chunk(N_MAIN_CHUNKS - 1, slabB, rowbufB, semSB, 0)
                for k in range(8):
                    pltpu.async_copy(
                        tail_tab.at[pl.ds(8 * k, 8), pl.ds(0, BLK)],
                        tslab.at[pl.ds(0, 8), pl.ds(k * BLK, BLK)], semB)
                for k in range(8):
                    pltpu.make_async_copy(
                        tail_tab.at[pl.ds(0, 8), pl.ds(0, BLK)],
                        tslab.at[pl.ds(0, 8), pl.ds(k * BLK, BLK)], semB).wait()
                s0 = gscal(sseg, N_MAIN_CHUNKS)
                abase = pl.multiple_of(gscal(sast, N_MAIN_CHUNKS), LANES)
                pt = extract(tslab, rowbufB, semSB, pt, s0, B, abase, MAIN_COLS, 0, BLK)
                drain_writes(pt, rowbufB, semSB)

    vg, ug = _scan_extract(tV, tU, tailV, tailU, sc, seg_c, ast_c, sx, seg_x, ast_x)

    @functools.partial(
        pl.kernel,
        out_type=jax.ShapeDtypeStruct((B,), jnp.float32),
        mesh=_make_mesh(),
        compiler_params=_params,
        scratch_types=[
            pltpu.VMEM((128, BLK), jnp.float32),
            pltpu.VMEM((128, BLK), jnp.float32),
            pltpu.VMEM((512,), jnp.float32),
            pltpu.VMEM((512,), jnp.int32),
            pltpu.VMEM((512,), jnp.int32),
            pltpu.SemaphoreType.DMA,
        ],
    )
    def _dot(vg_h, ug_h, rc_h, rx_h, o_h, vs, us, obuf, ridc, ridx, sem):
        wid = lax.axis_index("s") * 2 + lax.axis_index("c")
        iota = lax.iota(jnp.int32, LANES)
        base = wid * 512
        pltpu.sync_copy(rc_h.at[pl.ds(base, 512)], ridc)
        pltpu.sync_copy(rx_h.at[pl.ds(base, 512)], ridx)

        def quarter(q, carry):
            q0 = pl.multiple_of(q * 128, 128)
            cp1 = pltpu.async_copy(vg_h.at[ridc.at[pl.ds(q0, 128)]], vs, sem)
            cp2 = pltpu.async_copy(ug_h.at[ridx.at[pl.ds(q0, 128)]], us, sem)
            cp1.wait()
            cp2.wait()

            def blk(m, carry2):
                b0 = pl.multiple_of(m * LANES, LANES)
                tot = jnp.zeros((LANES,), jnp.float32)
                for r in range(LANES):
                    row = b0 + r
                    acc = vs[row, pl.ds(0, 16)] * us[row, pl.ds(0, 16)]
                    for j in range(1, DIM // LANES):
                        acc = acc + vs[row, pl.ds(16 * j, 16)] * us[row, pl.ds(16 * j, 16)]
                    tot = jnp.where(iota == r, jnp.sum(acc), tot)
                oq = pl.multiple_of(q * 128 + b0, LANES)
                obuf[pl.ds(oq, LANES)] = tot
                return carry2

            lax.fori_loop(0, 8, blk, 0)
            return carry

        lax.fori_loop(0, 4, quarter, 0)
        pltpu.sync_copy(obuf, o_h.at[pl.ds(base, 512)])

    return _dot(vg, ug, rowid_c, rowid_x)


# speedup vs baseline: 2.7265x; 1.4694x over previous
import functools

import jax
import jax.numpy as jnp
from jax import lax
from jax.experimental import pallas as pl
from jax.experimental.pallas import tpu as pltpu
from jax.experimental.pallas import tpu_sc as plsc

DIM = 64
LANES = 16
BLK = 128
CHUNK = 512
N_MAIN_CHUNKS = 1953
MAIN_COLS = N_MAIN_CHUNKS * CHUNK
NCHUNKS = N_MAIN_CHUNKS + 1
CHUNKS_PER_W = 61
B = 16384
GATH_ROWS = B + NCHUNKS * (LANES - 1) + 2 * LANES
COLSL = ((GATH_ROWS + 15) // 16) * 16

_params = pltpu.CompilerParams(needs_layout_passes=False, use_tc_tiling_on_sc=True)


def _make_mesh():
    return plsc.VectorSubcoreMesh(core_axis_name="c", subcore_axis_name="s")


def kernel(centers, contexts, in_embed, out_embed):
    V = in_embed.shape[0]
    ar = jnp.arange(NCHUNKS, dtype=jnp.int32)
    jj = jnp.arange(128, dtype=jnp.int32)

    def prep(idx):
        cid = idx // CHUNK
        cb = cid.reshape(128, 128)
        A3 = cb[:, :, None] == ar[None, None, :]
        Bh = A3.sum(axis=1, dtype=jnp.int32)
        hist = Bh.sum(axis=0)
        slots = ((hist + LANES - 1) // LANES) * LANES
        astart = (jnp.cumsum(slots) - slots).astype(jnp.int32)
        sast = (jnp.zeros((2048,), jnp.int32)
                .at[:NCHUNKS].set(astart)
                .at[NCHUNKS].set(jnp.sum(slots).astype(jnp.int32)))
        Cpref = jnp.cumsum(Bh, axis=0) - Bh
        M = (cb[:, :, None] == cb[:, None, :]) & (jj[None, None, :] < jj[None, :, None])
        rib = M.sum(axis=2, dtype=jnp.int32)
        cpg = Cpref.reshape(-1)[jj[:, None] * NCHUNKS + cb]
        rank = (cpg + rib).reshape(B)
        rowid = (astart[cid] + rank).astype(jnp.int32)
        return sast, rowid

    cvals = centers.astype(jnp.int32)
    xvals = contexts.astype(jnp.int32)
    ast_c, rowid_c = prep(cvals)
    ast_x, rowid_x = prep(xvals)

    tV = in_embed.T
    tU = out_embed.T
    tailV = jnp.pad(tV[:, MAIN_COLS:], ((0, 0), (0, BLK - (V - MAIN_COLS))))
    tailU = jnp.pad(tU[:, MAIN_COLS:], ((0, 0), (0, BLK - (V - MAIN_COLS))))

    colsl_t = jax.ShapeDtypeStruct((COLSL,), jnp.int32)

    @functools.partial(
        pl.kernel,
        out_type=(colsl_t, colsl_t),
        mesh=_make_mesh(),
        compiler_params=_params,
        scratch_types=[
            pltpu.VMEM((512,), jnp.int32),
            pltpu.VMEM((128,), jnp.int32),
            pltpu.VMEM((128,), jnp.int32),
            pltpu.VMEM((128,), jnp.int32),
            pltpu.VMEM((128,), jnp.int32),
            pltpu.SemaphoreType.DMA,
        ],
    )
    def _slotfill(cv_h, rc_h, xv_h, rx_h, cs_c, cs_x, vals, r0, r1, r2, r3, sem):
        wid = lax.axis_index("s") * 2 + lax.axis_index("c")
        base = wid * 512
        for v_h, rid_h, cs in ((cv_h, rc_h, cs_c), (xv_h, rx_h, cs_x)):
            pltpu.sync_copy(v_h.at[pl.ds(base, 512)], vals)
            for q, rq in enumerate((r0, r1, r2, r3)):
                pltpu.sync_copy(rid_h.at[pl.ds(base + q * 128, 128)], rq)
            cps = []
            for q, rq in enumerate((r0, r1, r2, r3)):
                cps.append(pltpu.async_copy(
                    vals.at[pl.ds(q * 128, 128)], cs.at[rq], sem))
            for cp in cps:
                cp.wait()

    colslot_c, colslot_x = _slotfill(cvals, rowid_c, xvals, rowid_x)

    gath = jax.ShapeDtypeStruct((GATH_ROWS, BLK), jnp.float32)

    @functools.partial(
        pl.kernel,
        out_type=(gath, gath),
        mesh=_make_mesh(),
        compiler_params=_params,
        scratch_types=[
            pltpu.VMEM((8, 8 * CHUNK), jnp.float32),
            pltpu.VMEM((8, 8 * CHUNK), jnp.float32),
            pltpu.VMEM((8, 8 * BLK), jnp.float32),
            pltpu.VMEM((COLSL,), jnp.int32),
            pltpu.VMEM((2048,), jnp.int32),
            pltpu.VMEM((LANES, BLK), jnp.float32),
            pltpu.VMEM((LANES, BLK), jnp.float32),
            pltpu.SemaphoreType.DMA,
            pltpu.SemaphoreType.DMA,
            pltpu.SemaphoreType.DMA,
            pltpu.SemaphoreType.DMA,
        ],
    )
    def _scan_extract(tv_h, tu_h, tlv_h, tlu_h, cs_c_h, ac_h, cs_x_h, ax_h,
                      vg, ug, slabA, slabB, tslab, colsl, sast,
                      rowbufA, rowbufB, semA, semB, semSA, semSB):
        wid = lax.axis_index("s") * 2 + lax.axis_index("c")
        iota = lax.iota(jnp.int32, LANES)
        sv8 = iota % 8
        k2v = iota // 8

        def gscal(ref, pos):
            return plsc.load_gather(ref, [jnp.zeros((LANES,), jnp.int32) + pos])[0]

        for tab, tail_tab, cs_h, ah, gout in (
            (tv_h, tlv_h, cs_c_h, ac_h, vg),
            (tu_h, tlu_h, cs_x_h, ax_h, ug),
        ):
            pltpu.sync_copy(cs_h, colsl)
            pltpu.sync_copy(ah, sast)

            def fire_stream(g, slab, sem):
                col0 = pl.multiple_of(g * CHUNK, CHUNK)
                for k in range(8):
                    pltpu.async_copy(
                        tab.at[pl.ds(8 * k, 8), pl.ds(col0, CHUNK)],
                        slab.at[pl.ds(0, 8), pl.ds(k * CHUNK, CHUNK)], sem)

            def wait_stream(slab, sem):
                for k in range(8):
                    pltpu.make_async_copy(
                        tab.at[pl.ds(0, 8), pl.ds(0, CHUNK)],
                        slab.at[pl.ds(0, 8), pl.ds(k * CHUNK, CHUNK)], sem).wait()

            def drain_writes(n, rowbuf, semS):
                def d(_, cc):
                    pltpu.make_async_copy(
                        gout.at[pl.ds(0, LANES), pl.ds(0, BLK)], rowbuf, semS).wait()
                    return cc
                lax.fori_loop(0, n, d, 0)

            def extract(slab, rowbuf, semS, pend, abase, aend, col0, base, mult):
                n_grp = (aend - abase) // LANES

                def grp(m, p):
                    drain_writes(p, rowbuf, semS)
                    pv = abase + m * LANES + iota
                    rvec = plsc.load_gather(colsl, [pv])
                    cvec = jnp.clip(rvec - col0, 0, mult - 1)
                    for l in range(LANES):
                        cl = cvec[l]
                        for j in range(DIM // LANES):
                            colv = base + (2 * j + k2v) * mult + cl
                            rowbuf[l, pl.ds(16 * j, 16)] = plsc.load_gather(slab, [sv8, colv])
                    arow = pl.multiple_of(abase + m * LANES, LANES)
                    pltpu.async_copy(
                        rowbuf, gout.at[pl.ds(arow, LANES), pl.ds(0, BLK)], semS)
                    return 1

                return lax.fori_loop(0, n_grp, grp, pend)

            def extract_chunk(g, slab, rowbuf, semS, pend):
                col0 = pl.multiple_of(g * CHUNK, CHUNK)
                abase = pl.multiple_of(gscal(sast, g), LANES)
                aend = gscal(sast, g + 1)
                return extract(slab, rowbuf, semS, pend, abase, aend, col0, 0, CHUNK)

            g_base = wid * CHUNKS_PER_W
            fire_stream(g_base, slabA, semA)

            def pair_body(p, carry):
                pa, pb = carry
                g0 = g_base + 2 * p
                fire_stream(g0 + 1, slabB, semB)
                wait_stream(slabA, semA)
                pa = extract_chunk(g0, slabA, rowbufA, semSA, pa)
                fire_stream(g0 + 2, slabA, semA)
                wait_stream(slabB, semB)
                pb = extract_chunk(g0 + 1, slabB, rowbufB, semSB, pb)
                return pa, pb

            pa, pb = lax.fori_loop(0, CHUNKS_PER_W // 2, pair_body, (0, 0))
            wait_stream(slabA, semA)
            pa = extract_chunk(g_base + CHUNKS_PER_W - 1, slabA, rowbufA, semSA, pa)
            drain_writes(pa, rowbufA, semSA)
            drain_writes(pb, rowbufB, semSB)

            @pl.when(wid == 31)
            def _tail():
                fire_stream(N_MAIN_CHUNKS - 1, slabB, semB)
                wait_stream(slabB, semB)
                pt = extract_chunk(N_MAIN_CHUNKS - 1, slabB, rowbufB, semSB, 0)
                for k in range(8):
                    pltpu.async_copy(
                        tail_tab.at[pl.ds(8 * k, 8), pl.ds(0, BLK)],
                        tslab.at[pl.ds(0, 8), pl.ds(k * BLK, BLK)], semB)
                for k in range(8):
                    pltpu.make_async_copy(
                        tail_tab.at[pl.ds(0, 8), pl.ds(0, BLK)],
                        tslab.at[pl.ds(0, 8), pl.ds(k * BLK, BLK)], semB).wait()
                abase = pl.multiple_of(gscal(sast, N_MAIN_CHUNKS), LANES)
                aend = gscal(sast, NCHUNKS)
                pt = extract(tslab, rowbufB, semSB, pt, abase, aend, MAIN_COLS, 0, BLK)
                drain_writes(pt, rowbufB, semSB)

    vg, ug = _scan_extract(tV, tU, tailV, tailU, colslot_c, ast_c, colslot_x, ast_x)

    @functools.partial(
        pl.kernel,
        out_type=jax.ShapeDtypeStruct((B,), jnp.float32),
        mesh=_make_mesh(),
        compiler_params=_params,
        scratch_types=[
            pltpu.VMEM((128, BLK), jnp.float32),
            pltpu.VMEM((128, BLK), jnp.float32),
            pltpu.VMEM((512,), jnp.float32),
            pltpu.VMEM((512,), jnp.int32),
            pltpu.VMEM((512,), jnp.int32),
            pltpu.SemaphoreType.DMA,
        ],
    )
    def _dot(vg_h, ug_h, rc_h, rx_h, o_h, vs, us, obuf, ridc, ridx, sem):
        wid = lax.axis_index("s") * 2 + lax.axis_index("c")
        iota = lax.iota(jnp.int32, LANES)
        base = wid * 512
        pltpu.sync_copy(rc_h.at[pl.ds(base, 512)], ridc)
        pltpu.sync_copy(rx_h.at[pl.ds(base, 512)], ridx)

        def quarter(q, carry):
            q0 = pl.multiple_of(q * 128, 128)
            cp1 = pltpu.async_copy(vg_h.at[ridc.at[pl.ds(q0, 128)]], vs, sem)
            cp2 = pltpu.async_copy(ug_h.at[ridx.at[pl.ds(q0, 128)]], us, sem)
            cp1.wait()
            cp2.wait()

            def blk(m, carry2):
                b0 = pl.multiple_of(m * LANES, LANES)
                tot = jnp.zeros((LANES,), jnp.float32)
                for r in range(LANES):
                    row = b0 + r
                    acc = vs[row, pl.ds(0, 16)] * us[row, pl.ds(0, 16)]
                    for j in range(1, DIM // LANES):
                        acc = acc + vs[row, pl.ds(16 * j, 16)] * us[row, pl.ds(16 * j, 16)]
                    tot = jnp.where(iota == r, jnp.sum(acc), tot)
                oq = pl.multiple_of(q * 128 + b0, LANES)
                obuf[pl.ds(oq, LANES)] = tot
                return carry2

            lax.fori_loop(0, 8, blk, 0)
            return carry

        lax.fori_loop(0, 4, quarter, 0)
        pltpu.sync_copy(obuf, o_h.at[pl.ds(base, 512)])

    return _dot(vg, ug, rowid_c, rowid_x)
